# baseline (device time: 101311 ns/iter reference)
import jax
import jax.numpy as jnp
from jax import lax
from jax.experimental import pallas as pl
from jax.experimental.pallas import tpu as pltpu

N_DEV = 8


def kernel(x, router_W, route_idx, expert_W, shared_W):
    n_tok, d_model = x.shape
    e_per, _, d_ff = expert_W.shape
    n_exp = N_DEV * e_per

    def body(x_ref, rw_ref, idx_ref, ew_ref, sw_ref, out_ref,
             comm_ref, send_sems, recv_sems):
        my = lax.axis_index("i")
        left = lax.rem(my + N_DEV - 1, N_DEV)
        right = lax.rem(my + 1, N_DEV)

        barrier_sem = pltpu.get_barrier_semaphore()
        for nbr in (left, right):
            pl.semaphore_signal(
                barrier_sem, inc=1,
                device_id=(nbr,), device_id_type=pl.DeviceIdType.MESH,
            )
        pl.semaphore_wait(barrier_sem, 2)

        xv = x_ref[...]
        scores = jnp.dot(xv, rw_ref[...], preferred_element_type=jnp.float32)
        s_max = jnp.max(scores, axis=-1, keepdims=True)
        ex = jnp.exp(scores - s_max)
        probs = ex / jnp.sum(ex, axis=-1, keepdims=True)
        route = idx_ref[...]
        eids = lax.broadcasted_iota(jnp.int32, (n_tok, n_exp), 1)
        p_sel = jnp.sum(jnp.where(route == eids, probs, 0.0),
                        axis=-1, keepdims=True)

        out_ref[...] = jnp.dot(xv, sw_ref[...],
                               preferred_element_type=jnp.float32)

        comm_ref[0] = ew_ref[...]

        fwd = [
            pltpu.make_async_remote_copy(
                src_ref=comm_ref.at[h],
                dst_ref=comm_ref.at[h + 1],
                send_sem=send_sems.at[h],
                recv_sem=recv_sems.at[h + 1],
                device_id=(right,),
                device_id_type=pl.DeviceIdType.MESH,
            )
            for h in range(N_DEV - 1)
        ]

        def compute_hop(h):
            src_dev = lax.rem(my - h + N_DEV, N_DEV)
            for k in range(e_per):
                e = src_dev * e_per + k
                gate = jnp.where(route == e, p_sel, 0.0)
                out_ref[...] += jnp.dot(
                    xv * gate, comm_ref[h, k],
                    preferred_element_type=jnp.float32,
                )

        fwd[0].start()
        compute_hop(0)
        for h in range(1, N_DEV):
            fwd[h - 1].wait_recv()
            if h < N_DEV - 1:
                fwd[h].start()
            compute_hop(h)
        for h in range(N_DEV - 1):
            fwd[h].wait_send()

    return pl.pallas_call(
        body,
        out_shape=jax.ShapeDtypeStruct((n_tok, d_ff), jnp.float32),
        in_specs=[pl.BlockSpec(memory_space=pltpu.VMEM)] * 5,
        out_specs=pl.BlockSpec(memory_space=pltpu.VMEM),
        scratch_shapes=[
            pltpu.VMEM((N_DEV, e_per, d_model, d_ff), jnp.float32),
            pltpu.SemaphoreType.DMA((N_DEV,)),
            pltpu.SemaphoreType.DMA((N_DEV,)),
        ],
        compiler_params=pltpu.CompilerParams(collective_id=0),
    )(x, router_W, route_idx, expert_W, shared_W)


# device time: 62071 ns/iter; 1.6322x vs baseline; 1.6322x over previous
import jax
import jax.numpy as jnp
from jax import lax
from jax.experimental import pallas as pl
from jax.experimental.pallas import tpu as pltpu

N_DEV = 8
CW_HOPS = 4
CCW_HOPS = 3


def kernel(x, router_W, route_idx, expert_W, shared_W):
    n_tok, d_model = x.shape
    e_per, _, d_ff = expert_W.shape
    n_exp = N_DEV * e_per

    def body(x_ref, rw_ref, idx_ref, ew_ref, sw_ref, out_ref,
             cw_ref, ccw_ref, cw_send, cw_recv, ccw_send, ccw_recv):
        my = lax.axis_index("i")
        left = lax.rem(my + N_DEV - 1, N_DEV)
        right = lax.rem(my + 1, N_DEV)

        barrier_sem = pltpu.get_barrier_semaphore()
        for nbr in (left, right):
            pl.semaphore_signal(
                barrier_sem, inc=1,
                device_id=(nbr,), device_id_type=pl.DeviceIdType.MESH,
            )
        pl.semaphore_wait(barrier_sem, 2)

        xv = x_ref[...]
        scores = jnp.dot(xv, rw_ref[...], preferred_element_type=jnp.float32)
        s_max = jnp.max(scores, axis=-1, keepdims=True)
        ex = jnp.exp(scores - s_max)
        probs = ex / jnp.sum(ex, axis=-1, keepdims=True)
        route = idx_ref[...]
        eids = lax.broadcasted_iota(jnp.int32, (n_tok, n_exp), 1)
        p_sel = jnp.sum(jnp.where(route == eids, probs, 0.0),
                        axis=-1, keepdims=True)

        def make_chain(comm_ref, send_sems, recv_sems, hops, target):
            return [
                pltpu.make_async_remote_copy(
                    src_ref=(ew_ref if h == 0 else comm_ref.at[h]),
                    dst_ref=comm_ref.at[h + 1],
                    send_sem=send_sems.at[h],
                    recv_sem=recv_sems.at[h + 1],
                    device_id=(target,),
                    device_id_type=pl.DeviceIdType.MESH,
                )
                for h in range(hops)
            ]

        cw_fwd = make_chain(cw_ref, cw_send, cw_recv, CW_HOPS, right)
        ccw_fwd = make_chain(ccw_ref, ccw_send, ccw_recv, CCW_HOPS, left)

        cw_fwd[0].start()
        ccw_fwd[0].start()

        def compute_block(block, src_dev):
            for k in range(e_per):
                e = src_dev * e_per + k
                gate = jnp.where(route == e, p_sel, 0.0)
                out_ref[...] += jnp.dot(
                    xv * gate, block[k],
                    preferred_element_type=jnp.float32,
                )

        out_ref[...] = jnp.dot(xv, sw_ref[...],
                               preferred_element_type=jnp.float32)
        compute_block(ew_ref, my)

        for r in range(1, CW_HOPS + 1):
            cw_fwd[r - 1].wait_recv()
            if r < CW_HOPS:
                cw_fwd[r].start()
            if r <= CCW_HOPS:
                ccw_fwd[r - 1].wait_recv()
                if r < CCW_HOPS:
                    ccw_fwd[r].start()
            compute_block(cw_ref.at[r], lax.rem(my - r + N_DEV, N_DEV))
            if r <= CCW_HOPS:
                compute_block(ccw_ref.at[r], lax.rem(my + r, N_DEV))

        for d in cw_fwd + ccw_fwd:
            d.wait_send()

    return pl.pallas_call(
        body,
        out_shape=jax.ShapeDtypeStruct((n_tok, d_ff), jnp.float32),
        in_specs=[pl.BlockSpec(memory_space=pltpu.VMEM)] * 5,
        out_specs=pl.BlockSpec(memory_space=pltpu.VMEM),
        scratch_shapes=[
            pltpu.VMEM((CW_HOPS + 1, e_per, d_model, d_ff), jnp.float32),
            pltpu.VMEM((CCW_HOPS + 1, e_per, d_model, d_ff), jnp.float32),
            pltpu.SemaphoreType.DMA((CW_HOPS,)),
            pltpu.SemaphoreType.DMA((CW_HOPS + 1,)),
            pltpu.SemaphoreType.DMA((CCW_HOPS,)),
            pltpu.SemaphoreType.DMA((CCW_HOPS + 1,)),
        ],
        compiler_params=pltpu.CompilerParams(collective_id=0),
    )(x, router_W, route_idx, expert_W, shared_W)


# device time: 39624 ns/iter; 2.5568x vs baseline; 1.5665x over previous
import jax
import jax.numpy as jnp
from jax import lax
from jax.experimental import pallas as pl
from jax.experimental.pallas import tpu as pltpu

N_DEV = 8
CW_HOPS = 4
CCW_HOPS = 3


def kernel(x, router_W, route_idx, expert_W, shared_W):
    n_tok, d_model = x.shape
    e_per, _, d_ff = expert_W.shape
    n_exp = N_DEV * e_per

    def body(x_ref, rw_ref, idx_ref, ew_ref, sw_ref, out_ref,
             ewb_ref, cw_ref, ccw_ref, cw_send, cw_recv, ccw_send, ccw_recv):
        my = lax.axis_index("i")
        left = lax.rem(my + N_DEV - 1, N_DEV)
        right = lax.rem(my + 1, N_DEV)

        barrier_sem = pltpu.get_barrier_semaphore()
        for nbr in (left, right):
            pl.semaphore_signal(
                barrier_sem, inc=1,
                device_id=(nbr,), device_id_type=pl.DeviceIdType.MESH,
            )
        pl.semaphore_wait(barrier_sem, 2)

        xv = x_ref[...]
        ewb_ref[...] = ew_ref[...].astype(jnp.bfloat16)
        scores = jnp.dot(xv, rw_ref[...], preferred_element_type=jnp.float32)
        s_max = jnp.max(scores, axis=-1, keepdims=True)
        ex = jnp.exp(scores - s_max)
        probs = ex / jnp.sum(ex, axis=-1, keepdims=True)
        route = idx_ref[...]
        eids = lax.broadcasted_iota(jnp.int32, (n_tok, n_exp), 1)
        p_sel = jnp.sum(jnp.where(route == eids, probs, 0.0),
                        axis=-1, keepdims=True)

        def make_chain(comm_ref, send_sems, recv_sems, hops, target):
            return [
                pltpu.make_async_remote_copy(
                    src_ref=(ewb_ref if h == 0 else comm_ref.at[h]),
                    dst_ref=comm_ref.at[h + 1],
                    send_sem=send_sems.at[h],
                    recv_sem=recv_sems.at[h + 1],
                    device_id=(target,),
                    device_id_type=pl.DeviceIdType.MESH,
                )
                for h in range(hops)
            ]

        cw_fwd = make_chain(cw_ref, cw_send, cw_recv, CW_HOPS, right)
        ccw_fwd = make_chain(ccw_ref, ccw_send, ccw_recv, CCW_HOPS, left)

        cw_fwd[0].start()
        ccw_fwd[0].start()

        def compute_block(block, src_dev):
            for k in range(e_per):
                e = src_dev * e_per + k
                gate = jnp.where(route == e, p_sel, 0.0)
                out_ref[...] += jnp.dot(
                    (xv * gate).astype(jnp.bfloat16), block[k],
                    preferred_element_type=jnp.float32,
                )

        out_ref[...] = jnp.dot(
            xv.astype(jnp.bfloat16), sw_ref[...].astype(jnp.bfloat16),
            preferred_element_type=jnp.float32,
        )
        compute_block(ewb_ref, my)

        for r in range(1, CW_HOPS + 1):
            cw_fwd[r - 1].wait_recv()
            if r < CW_HOPS:
                cw_fwd[r].start()
            if r <= CCW_HOPS:
                ccw_fwd[r - 1].wait_recv()
                if r < CCW_HOPS:
                    ccw_fwd[r].start()
            compute_block(cw_ref.at[r], lax.rem(my - r + N_DEV, N_DEV))
            if r <= CCW_HOPS:
                compute_block(ccw_ref.at[r], lax.rem(my + r, N_DEV))

        for d in cw_fwd + ccw_fwd:
            d.wait_send()

    return pl.pallas_call(
        body,
        out_shape=jax.ShapeDtypeStruct((n_tok, d_ff), jnp.float32),
        in_specs=[pl.BlockSpec(memory_space=pltpu.VMEM)] * 5,
        out_specs=pl.BlockSpec(memory_space=pltpu.VMEM),
        scratch_shapes=[
            pltpu.VMEM((e_per, d_model, d_ff), jnp.bfloat16),
            pltpu.VMEM((CW_HOPS + 1, e_per, d_model, d_ff), jnp.bfloat16),
            pltpu.VMEM((CCW_HOPS + 1, e_per, d_model, d_ff), jnp.bfloat16),
            pltpu.SemaphoreType.DMA((CW_HOPS,)),
            pltpu.SemaphoreType.DMA((CW_HOPS + 1,)),
            pltpu.SemaphoreType.DMA((CCW_HOPS,)),
            pltpu.SemaphoreType.DMA((CCW_HOPS + 1,)),
        ],
        compiler_params=pltpu.CompilerParams(collective_id=0),
    )(x, router_W, route_idx, expert_W, shared_W)


# device time: 35151 ns/iter; 2.8822x vs baseline; 1.1273x over previous
import jax
import jax.numpy as jnp
from jax import lax
from jax.experimental import pallas as pl
from jax.experimental.pallas import tpu as pltpu

N_DEV = 8
CW_HOPS = 4
CCW_HOPS = 3


def kernel(x, router_W, route_idx, expert_W, shared_W):
    n_tok, d_model = x.shape
    e_per, _, d_ff = expert_W.shape
    n_exp = N_DEV * e_per

    def body(x_ref, rw_ref, idx_ref, ew_ref, sw_ref, out_ref,
             ewb_ref, cw_ref, ccw_ref, cw_send, cw_recv, ccw_send, ccw_recv):
        my = lax.axis_index("i")
        left = lax.rem(my + N_DEV - 1, N_DEV)
        right = lax.rem(my + 1, N_DEV)

        barrier_sem = pltpu.get_barrier_semaphore()
        for nbr in (left, right):
            pl.semaphore_signal(
                barrier_sem, inc=1,
                device_id=(nbr,), device_id_type=pl.DeviceIdType.MESH,
            )
        pl.semaphore_wait(barrier_sem, 2)

        xv = x_ref[...]
        ewb_ref[...] = ew_ref[...].astype(jnp.bfloat16)
        scores = jnp.dot(xv, rw_ref[...], preferred_element_type=jnp.float32)
        s_max = jnp.max(scores, axis=-1, keepdims=True)
        ex = jnp.exp(scores - s_max)
        probs = ex / jnp.sum(ex, axis=-1, keepdims=True)
        route = idx_ref[...]
        eids = lax.broadcasted_iota(jnp.int32, (n_tok, n_exp), 1)
        p_sel = jnp.sum(jnp.where(route == eids, probs, 0.0),
                        axis=-1, keepdims=True)

        def make_chain(comm_ref, send_sems, recv_sems, hops, target):
            return [
                [
                    pltpu.make_async_remote_copy(
                        src_ref=(ewb_ref.at[k] if h == 0
                                 else comm_ref.at[h, k]),
                        dst_ref=comm_ref.at[h + 1, k],
                        send_sem=send_sems.at[h, k],
                        recv_sem=recv_sems.at[h + 1, k],
                        device_id=(target,),
                        device_id_type=pl.DeviceIdType.MESH,
                    )
                    for k in range(e_per)
                ]
                for h in range(hops)
            ]

        cw_fwd = make_chain(cw_ref, cw_send, cw_recv, CW_HOPS, right)
        ccw_fwd = make_chain(ccw_ref, ccw_send, ccw_recv, CCW_HOPS, left)

        for k in range(e_per):
            cw_fwd[0][k].start()
            ccw_fwd[0][k].start()

        def compute_expert(w_k, src_dev, k):
            e = src_dev * e_per + k
            gate = jnp.where(route == e, p_sel, 0.0)
            out_ref[...] += jnp.dot(
                (xv * gate).astype(jnp.bfloat16), w_k,
                preferred_element_type=jnp.float32,
            )

        out_ref[...] = jnp.dot(
            xv.astype(jnp.bfloat16), sw_ref[...].astype(jnp.bfloat16),
            preferred_element_type=jnp.float32,
        )
        for k in range(e_per):
            compute_expert(ewb_ref[k], my, k)

        def recv_and_forward(chain, r, hops):
            for k in range(e_per):
                chain[r - 1][k].wait_recv()
                if r < hops:
                    chain[r][k].start()

        for r in range(1, CW_HOPS + 1):
            recv_and_forward(cw_fwd, r, CW_HOPS)
            if r <= CCW_HOPS:
                recv_and_forward(ccw_fwd, r, CCW_HOPS)
            for k in range(e_per):
                compute_expert(cw_ref[r, k],
                               lax.rem(my - r + N_DEV, N_DEV), k)
            if r <= CCW_HOPS:
                for k in range(e_per):
                    compute_expert(ccw_ref[r, k], lax.rem(my + r, N_DEV), k)

        for chain in (cw_fwd, ccw_fwd):
            for hop in chain:
                for d in hop:
                    d.wait_send()

    return pl.pallas_call(
        body,
        out_shape=jax.ShapeDtypeStruct((n_tok, d_ff), jnp.float32),
        in_specs=[pl.BlockSpec(memory_space=pltpu.VMEM)] * 5,
        out_specs=pl.BlockSpec(memory_space=pltpu.VMEM),
        scratch_shapes=[
            pltpu.VMEM((e_per, d_model, d_ff), jnp.bfloat16),
            pltpu.VMEM((CW_HOPS + 1, e_per, d_model, d_ff), jnp.bfloat16),
            pltpu.VMEM((CCW_HOPS + 1, e_per, d_model, d_ff), jnp.bfloat16),
            pltpu.SemaphoreType.DMA((CW_HOPS, e_per)),
            pltpu.SemaphoreType.DMA((CW_HOPS + 1, e_per)),
            pltpu.SemaphoreType.DMA((CCW_HOPS, e_per)),
            pltpu.SemaphoreType.DMA((CCW_HOPS + 1, e_per)),
        ],
        compiler_params=pltpu.CompilerParams(collective_id=0),
    )(x, router_W, route_idx, expert_W, shared_W)
